# baseline (speedup 1.0000x reference)
import jax
import jax.numpy as jnp
from jax.experimental import pallas as pl
from jax.experimental.pallas import tpu as pltpu

B, S, H, D = 8, 2048, 8, 128


def _kv_body(len_ref, c_in, tok, c_out):
    b = pl.program_id(0)
    c_out[...] = c_in[...]
    l = len_ref[b]
    c_out[0, pl.ds(l, 1)] = tok[pl.ds(b, 1), 0]


def _append_one(cache, token, lengths):
    out_sds = jax.ShapeDtypeStruct((B, S, H, D), jnp.float32)
    cache_spec = pl.BlockSpec((1, S, H, D), lambda b: (b, 0, 0, 0))
    token_spec = pl.BlockSpec((B, 1, H, D), lambda b: (0, 0, 0, 0))
    return pl.pallas_call(
        _kv_body,
        grid=(B,),
        in_specs=[
            pl.BlockSpec(memory_space=pltpu.SMEM),
            cache_spec,
            token_spec,
        ],
        out_specs=cache_spec,
        out_shape=out_sds,
        compiler_params=pltpu.CompilerParams(
            dimension_semantics=("parallel",),
            vmem_limit_bytes=60 * 1024 * 1024,
        ),
    )(lengths, cache, token)


def kernel(cached_key, cached_value, key_token, value_token, lengths):
    new_key = _append_one(cached_key, key_token, lengths)
    new_value = _append_one(cached_value, value_token, lengths)
    return (new_key, new_value)
